# baseline (speedup 1.0000x reference)
import functools

import jax
import jax.numpy as jnp
import numpy as np
from jax.experimental import pallas as pl

_VOCAB = 4096
_B = 64
_C = 32
_HW = 16
_S = _HW * _HW
_N_FULL = _B * _S
_PNS = (1, 2, 4, 8, 16)
_PI = (0, 1, 1, 2)
_F32 = jnp.float32
_HIGH = jax.lax.Precision.HIGHEST


def _pool_mat_t(pn: int) -> np.ndarray:
    k = _HW // pn
    p1 = np.zeros((pn, _HW), np.float32)
    for p in range(pn):
        p1[p, p * k:(p + 1) * k] = 1.0 / k
    p2 = np.kron(p1, p1)
    return np.ascontiguousarray(p2.T)


@functools.lru_cache(maxsize=None)
def _upsample_mat_t(pn: int):
    a = jax.image.resize(jnp.eye(pn, dtype=_F32), (_HW, pn), method="bicubic")
    u = jnp.kron(a, a)
    return u.T


def _prep_body(emb_ref, en_ref):
    emb = emb_ref[...]
    norm = jnp.sqrt(jnp.sum(emb * emb, axis=1, keepdims=True))
    en_ref[...] = emb / jnp.maximum(norm, 1e-12)


def _pool_body(fr_ref, pt_ref, out_ref):
    out_ref[...] = jax.lax.dot(fr_ref[...], pt_ref[...], precision=_HIGH)


def _pool1_body(fr_ref, out_ref):
    out_ref[...] = jnp.sum(fr_ref[...], axis=1, keepdims=True) * (1.0 / _S)


def _quant_body(rest_ref, ent_ref, emb_ref, h_ref, hits_ref, *, chunk, want_h):
    rest = rest_ref[...]
    norm = jnp.sqrt(jnp.sum(rest * rest, axis=1, keepdims=True))
    rn = rest / jnp.maximum(norm, 1e-12)
    sims = jax.lax.dot(rn, ent_ref[...], precision=_HIGH)
    m = jnp.max(sims, axis=1, keepdims=True)
    iota = jax.lax.broadcasted_iota(jnp.int32, (chunk, _VOCAB), 1)
    idx = jnp.min(jnp.where(sims == m, iota, _VOCAB), axis=1, keepdims=True)
    onehot = (iota == idx).astype(_F32)
    if want_h:
        h_ref[...] = jax.lax.dot(onehot, emb_ref[...], precision=_HIGH)
    rows = jax.lax.broadcasted_iota(jnp.int32, (8, chunk), 0)
    cols = jax.lax.broadcasted_iota(jnp.int32, (8, chunk), 1)
    w8 = (cols // (chunk // 8) == rows).astype(_F32) if chunk >= 8 else (
        (rows == 0).astype(_F32))
    part = jax.lax.dot(w8, onehot, precision=_HIGH)

    @pl.when(pl.program_id(0) == 0)
    def _init():
        hits_ref[...] = jnp.zeros_like(hits_ref)

    hits_ref[...] += part


def _up_body(hcf_ref, ut_ref, out_ref):
    out_ref[...] = jax.lax.dot(hcf_ref[...], ut_ref[...], precision=_HIGH)


def _up1_body(hcf_ref, ut_ref, out_ref):
    out_ref[...] = hcf_ref[...] * ut_ref[...]


def _shift_rows(x, s, zeros):
    if s == 0:
        return x
    if s > 0:
        return jnp.concatenate([x[s:, :], zeros[:s, :]], axis=0)
    return jnp.concatenate([zeros[:(-s), :], x[:s, :]], axis=0)


_CONV_ROWS = 2048


def _conv_body(hup_ref, fr_ref, w9_ref, bias_ref, out_ref):
    x = hup_ref[...]
    zeros = jnp.zeros_like(x)
    ii = jax.lax.broadcasted_iota(jnp.int32, (_CONV_ROWS, 1), 0)
    w = ii & (_HW - 1)
    h = (ii >> 4) & (_HW - 1)
    y = jnp.broadcast_to(bias_ref[0:1, :], (_CONV_ROWS, _C)).astype(_F32)
    t = 0
    for dh in (-1, 0, 1):
        for dw in (-1, 0, 1):
            xs = _shift_rows(x, dh * _HW + dw, zeros)
            okh = jnp.logical_and(h + dh >= 0, h + dh < _HW)
            okw = jnp.logical_and(w + dw >= 0, w + dw < _HW)
            xm = jnp.where(jnp.logical_and(okh, okw), xs, 0.0)
            wt = w9_ref[t * _C:(t + 1) * _C, :].astype(jnp.bfloat16)
            y = y + jax.lax.dot(xm.astype(jnp.bfloat16), wt,
                                preferred_element_type=_F32)
            t += 1
    out_ref[...] = fr_ref[...] - 0.5 * x - 0.5 * y


def _ppl_body(hits_ref, f_ref, fhat_ref, loss_ref, ppl_ref):
    hits = jnp.sum(hits_ref[...], axis=0, keepdims=True)
    total = jnp.sum(hits)
    avg = hits / jnp.maximum(total, 1.0)
    ent = jnp.sum(avg * jnp.log(avg + 1e-10))
    ppl = jnp.exp(-ent)
    f = f_ref[...]
    loss = 6.25 * (jnp.sum(f * f) / (_B * _C * _S))
    fhat_ref[...] = jnp.zeros_like(fhat_ref)
    loss_ref[...] = jnp.full((8, 128), loss, _F32)
    ppl_ref[...] = jnp.full((8, 128), ppl, _F32)


def _vmem(shape):
    return pl.BlockSpec(shape, lambda *_: tuple(0 for _ in shape))


def _call(body, out_shapes, *args):
    specs = [pl.BlockSpec(a.shape, None) for a in args]
    outs = [pl.BlockSpec(s, None) for s in out_shapes]
    return pl.pallas_call(
        body,
        out_shape=[jax.ShapeDtypeStruct(s, _F32) for s in out_shapes],
        in_specs=specs,
        out_specs=outs,
    )(*args)


def _quant_call(rest_nc, en_t, emb, want_h):
    n = rest_nc.shape[0]
    chunk = min(n, 512)
    grid = n // chunk
    body = functools.partial(_quant_body, chunk=chunk, want_h=want_h)
    out_shapes = [jax.ShapeDtypeStruct((n, _C), _F32),
                  jax.ShapeDtypeStruct((8, _VOCAB), _F32)]
    out_specs = [pl.BlockSpec((chunk, _C), lambda i: (i, 0)),
                 pl.BlockSpec((8, _VOCAB), lambda i: (0, 0))]
    h, hits = pl.pallas_call(
        body,
        grid=(grid,),
        out_shape=out_shapes,
        in_specs=[pl.BlockSpec((chunk, _C), lambda i: (i, 0)),
                  pl.BlockSpec((_C, _VOCAB), lambda i: (0, 0)),
                  pl.BlockSpec((_VOCAB, _C), lambda i: (0, 0))],
        out_specs=out_specs,
    )(rest_nc, en_t, emb)
    return h, hits


def _cf_from_hw(x_hw):
    return x_hw.reshape(_B, _S, _C).transpose(0, 2, 1).reshape(_B * _C, _S)


def _hw_from_cf(x_cf):
    return x_cf.reshape(_B, _C, _S).transpose(0, 2, 1).reshape(_N_FULL, _C)


def _nc_from_cf(x_cf, sq):
    return x_cf.reshape(_B, _C, sq).transpose(0, 2, 1).reshape(_B * sq, _C)


def _cf_from_nc(x_nc, sq):
    return x_nc.reshape(_B, sq, _C).transpose(0, 2, 1).reshape(_B * _C, sq)


def kernel(f_BChw, emb_weight, phi_w, phi_b):
    f = f_BChw.astype(_F32)
    f_cf = f.reshape(_B * _C, _S)
    fr_hw = f.transpose(0, 2, 3, 1).reshape(_N_FULL, _C)

    (en,) = _call(_prep_body, [(_VOCAB, _C)], emb_weight)
    en_t = en.T

    hits_list = []
    for si, pn in enumerate(_PNS):
        sq = pn * pn
        last = si == len(_PNS) - 1
        if last:
            rest_nc = fr_hw
        else:
            fr_cf = f_cf if si == 0 else _cf_from_hw(fr_hw)
            if pn == 1:
                (rest_cf,) = _call(_pool1_body, [(_B * _C, 1)], fr_cf)
            else:
                pt = jnp.asarray(_pool_mat_t(pn))
                (rest_cf,) = _call(_pool_body, [(_B * _C, sq)], fr_cf, pt)
            rest_nc = _nc_from_cf(rest_cf, sq)

        h_nc, hits = _quant_call(rest_nc, en_t, emb_weight, want_h=not last)
        hits_list.append(hits)
        if last:
            break

        h_cf = _cf_from_nc(h_nc, sq)
        ut = _upsample_mat_t(pn)
        if pn == 1:
            (hup_cf,) = _call(_up1_body, [(_B * _C, _S)], h_cf, ut)
        else:
            (hup_cf,) = _call(_up_body, [(_B * _C, _S)], h_cf, ut)
        hup_hw = _hw_from_cf(hup_cf)

        pi = _PI[si]
        w9 = jnp.concatenate(
            [phi_w[pi, :, :, a, b].T for a in range(3) for b in range(3)],
            axis=0)
        bias8 = jnp.broadcast_to(phi_b[pi][None, :], (8, _C))
        (fr_hw,) = pl.pallas_call(
            _conv_body,
            grid=(_N_FULL // _CONV_ROWS,),
            out_shape=[jax.ShapeDtypeStruct((_N_FULL, _C), _F32)],
            in_specs=[pl.BlockSpec((_CONV_ROWS, _C), lambda i: (i, 0)),
                      pl.BlockSpec((_CONV_ROWS, _C), lambda i: (i, 0)),
                      pl.BlockSpec((288, _C), lambda i: (0, 0)),
                      pl.BlockSpec((8, _C), lambda i: (0, 0))],
            out_specs=[pl.BlockSpec((_CONV_ROWS, _C), lambda i: (i, 0))],
        )(hup_hw, fr_hw, w9, bias8)

    hits_all = jnp.concatenate(hits_list, axis=0)
    fhat_cf, loss_t, ppl_t = _call(
        _ppl_body, [(_B * _C, _S), (8, 128), (8, 128)], hits_all, f_cf)

    f_hat = fhat_cf.reshape(_B, _C, _HW, _HW)
    return (f_hat, loss_t[0, 0], ppl_t[0, 0])


# speedup vs baseline: 2.4575x; 2.4575x over previous
import functools

import jax
import jax.numpy as jnp
import numpy as np
from jax.experimental import pallas as pl

_VOCAB = 4096
_B = 64
_C = 32
_HW = 16
_S = _HW * _HW
_N_FULL = _B * _S
_PNS = (1, 2, 4, 8, 16)
_PI = (0, 1, 1, 2)
_F32 = jnp.float32
_HIGH = jax.lax.Precision.HIGHEST


def _pool_mat_t(pn: int) -> np.ndarray:
    k = _HW // pn
    p1 = np.zeros((pn, _HW), np.float32)
    for p in range(pn):
        p1[p, p * k:(p + 1) * k] = 1.0 / k
    p2 = np.kron(p1, p1)
    return np.ascontiguousarray(p2.T)


@functools.lru_cache(maxsize=None)
def _upsample_mat_t(pn: int):
    a = jax.image.resize(jnp.eye(pn, dtype=_F32), (_HW, pn), method="bicubic")
    u = jnp.kron(a, a)
    return u.T


def _prep_body(emb_ref, en_ref, hilo_ref):
    emb = emb_ref[...]
    norm = jnp.sqrt(jnp.sum(emb * emb, axis=1, keepdims=True))
    en_ref[...] = emb / jnp.maximum(norm, 1e-12)
    hi = emb.astype(jnp.bfloat16)
    lo = (emb - hi.astype(_F32)).astype(jnp.bfloat16)
    hilo_ref[...] = jnp.concatenate([hi, lo], axis=1)


def _prep2_body(ent_ref, b96_ref):
    ent = ent_ref[...]
    hi = ent.astype(jnp.bfloat16)
    lo = (ent - hi.astype(_F32)).astype(jnp.bfloat16)
    b96_ref[...] = jnp.concatenate([hi, lo, hi], axis=0)


def _pool_body(fr_ref, pt_ref, out_ref):
    out_ref[...] = jax.lax.dot(fr_ref[...], pt_ref[...], precision=_HIGH)


def _pool1_body(fr_ref, out_ref):
    out_ref[...] = jnp.sum(fr_ref[...], axis=1, keepdims=True) * (1.0 / _S)


def _quant_body(rest_ref, b96_ref, hilo_ref, h_ref, hits_ref, *,
                chunk, want_h):
    rest = rest_ref[...]
    rh = rest.astype(jnp.bfloat16)
    rl = (rest - rh.astype(_F32)).astype(jnp.bfloat16)
    a96 = jnp.concatenate([rh, rh, rl], axis=1)
    sims = jax.lax.dot(a96, b96_ref[...], preferred_element_type=_F32)
    m = jnp.max(sims, axis=1, keepdims=True)
    onehot = (sims == m).astype(jnp.bfloat16)
    if want_h:
        hl = jax.lax.dot(onehot, hilo_ref[...], preferred_element_type=_F32)
        h_ref[...] = hl[:, :_C] + hl[:, _C:]
    rows = jax.lax.broadcasted_iota(jnp.int32, (8, chunk), 0)
    cols = jax.lax.broadcasted_iota(jnp.int32, (8, chunk), 1)
    w8 = (cols // (chunk // 8) == rows) if chunk >= 8 else (rows == 0)
    part = jax.lax.dot(w8.astype(jnp.bfloat16), onehot,
                       preferred_element_type=_F32)

    @pl.when(pl.program_id(0) == 0)
    def _init():
        hits_ref[...] = jnp.zeros_like(hits_ref)

    hits_ref[...] += part


def _up_body(hcf_ref, ut_ref, out_ref):
    out_ref[...] = jax.lax.dot(hcf_ref[...], ut_ref[...], precision=_HIGH)


def _up1_body(hcf_ref, ut_ref, out_ref):
    out_ref[...] = hcf_ref[...] * ut_ref[...]


def _shift_rows(x, s, zeros):
    if s == 0:
        return x
    if s > 0:
        return jnp.concatenate([x[s:, :], zeros[:s, :]], axis=0)
    return jnp.concatenate([zeros[:(-s), :], x[:s, :]], axis=0)


_CONV_ROWS = 2048


def _conv_body(hup_ref, fr_ref, w9_ref, bias_ref, out_ref):
    x = hup_ref[...]
    zeros = jnp.zeros_like(x)
    ii = jax.lax.broadcasted_iota(jnp.int32, (_CONV_ROWS, 1), 0)
    w = ii & (_HW - 1)
    h = (ii >> 4) & (_HW - 1)
    y = jnp.broadcast_to(bias_ref[0:1, :], (_CONV_ROWS, _C)).astype(_F32)
    t = 0
    for dh in (-1, 0, 1):
        for dw in (-1, 0, 1):
            xs = _shift_rows(x, dh * _HW + dw, zeros)
            okh = jnp.logical_and(h + dh >= 0, h + dh < _HW)
            okw = jnp.logical_and(w + dw >= 0, w + dw < _HW)
            xm = jnp.where(jnp.logical_and(okh, okw), xs, 0.0)
            wt = w9_ref[t * _C:(t + 1) * _C, :].astype(jnp.bfloat16)
            y = y + jax.lax.dot(xm.astype(jnp.bfloat16), wt,
                                preferred_element_type=_F32)
            t += 1
    out_ref[...] = fr_ref[...] - 0.5 * x - 0.5 * y


def _ppl_body(hits_ref, f_ref, fhat_ref, loss_ref, ppl_ref):
    hits = jnp.sum(hits_ref[...], axis=0, keepdims=True)
    total = jnp.sum(hits)
    avg = hits / jnp.maximum(total, 1.0)
    ent = jnp.sum(avg * jnp.log(avg + 1e-10))
    ppl = jnp.exp(-ent)
    f = f_ref[...]
    loss = 6.25 * (jnp.sum(f * f) / (_B * _C * _S))
    fhat_ref[...] = jnp.zeros_like(fhat_ref)
    loss_ref[...] = jnp.full((8, 128), loss, _F32)
    ppl_ref[...] = jnp.full((8, 128), ppl, _F32)


def _vmem(shape):
    return pl.BlockSpec(shape, lambda *_: tuple(0 for _ in shape))


def _call(body, out_shapes, *args):
    specs = [pl.BlockSpec(a.shape, None) for a in args]
    outs = [pl.BlockSpec(s if isinstance(s, tuple) else s.shape, None)
            for s in out_shapes]
    return pl.pallas_call(
        body,
        out_shape=[jax.ShapeDtypeStruct(s, _F32) if isinstance(s, tuple) else s
                   for s in out_shapes],
        in_specs=specs,
        out_specs=outs,
    )(*args)


def _quant_call(rest_nc, b96, hilo, want_h):
    n = rest_nc.shape[0]
    chunk = min(n, 1024)
    grid = n // chunk
    body = functools.partial(_quant_body, chunk=chunk, want_h=want_h)
    out_shapes = [jax.ShapeDtypeStruct((n, _C), _F32),
                  jax.ShapeDtypeStruct((8, _VOCAB), _F32)]
    out_specs = [pl.BlockSpec((chunk, _C), lambda i: (i, 0)),
                 pl.BlockSpec((8, _VOCAB), lambda i: (0, 0))]
    h, hits = pl.pallas_call(
        body,
        grid=(grid,),
        out_shape=out_shapes,
        in_specs=[pl.BlockSpec((chunk, _C), lambda i: (i, 0)),
                  pl.BlockSpec((96, _VOCAB), lambda i: (0, 0)),
                  pl.BlockSpec((_VOCAB, 2 * _C), lambda i: (0, 0))],
        out_specs=out_specs,
    )(rest_nc, b96, hilo)
    return h, hits


def _cf_from_hw(x_hw):
    return x_hw.reshape(_B, _S, _C).transpose(0, 2, 1).reshape(_B * _C, _S)


def _hw_from_cf(x_cf):
    return x_cf.reshape(_B, _C, _S).transpose(0, 2, 1).reshape(_N_FULL, _C)


def _nc_from_cf(x_cf, sq):
    return x_cf.reshape(_B, _C, sq).transpose(0, 2, 1).reshape(_B * sq, _C)


def _cf_from_nc(x_nc, sq):
    return x_nc.reshape(_B, sq, _C).transpose(0, 2, 1).reshape(_B * _C, sq)


def kernel(f_BChw, emb_weight, phi_w, phi_b):
    f = f_BChw.astype(_F32)
    f_cf = f.reshape(_B * _C, _S)
    fr_hw = f.transpose(0, 2, 3, 1).reshape(_N_FULL, _C)

    en, hilo = _call(
        _prep_body,
        [(_VOCAB, _C),
         jax.ShapeDtypeStruct((_VOCAB, 2 * _C), jnp.bfloat16)],
        emb_weight)
    (b96,) = _call(
        _prep2_body,
        [jax.ShapeDtypeStruct((96, _VOCAB), jnp.bfloat16)],
        en.T)

    hits_list = []
    for si, pn in enumerate(_PNS):
        sq = pn * pn
        last = si == len(_PNS) - 1
        if last:
            rest_nc = fr_hw
        else:
            fr_cf = f_cf if si == 0 else _cf_from_hw(fr_hw)
            if pn == 1:
                (rest_cf,) = _call(_pool1_body, [(_B * _C, 1)], fr_cf)
            else:
                pt = jnp.asarray(_pool_mat_t(pn))
                (rest_cf,) = _call(_pool_body, [(_B * _C, sq)], fr_cf, pt)
            rest_nc = _nc_from_cf(rest_cf, sq)

        h_nc, hits = _quant_call(rest_nc, b96, hilo, want_h=not last)
        hits_list.append(hits)
        if last:
            break

        h_cf = _cf_from_nc(h_nc, sq)
        ut = _upsample_mat_t(pn)
        if pn == 1:
            (hup_cf,) = _call(_up1_body, [(_B * _C, _S)], h_cf, ut)
        else:
            (hup_cf,) = _call(_up_body, [(_B * _C, _S)], h_cf, ut)
        hup_hw = _hw_from_cf(hup_cf)

        pi = _PI[si]
        w9 = jnp.concatenate(
            [phi_w[pi, :, :, a, b].T for a in range(3) for b in range(3)],
            axis=0)
        bias8 = jnp.broadcast_to(phi_b[pi][None, :], (8, _C))
        (fr_hw,) = pl.pallas_call(
            _conv_body,
            grid=(_N_FULL // _CONV_ROWS,),
            out_shape=[jax.ShapeDtypeStruct((_N_FULL, _C), _F32)],
            in_specs=[pl.BlockSpec((_CONV_ROWS, _C), lambda i: (i, 0)),
                      pl.BlockSpec((_CONV_ROWS, _C), lambda i: (i, 0)),
                      pl.BlockSpec((288, _C), lambda i: (0, 0)),
                      pl.BlockSpec((8, _C), lambda i: (0, 0))],
            out_specs=[pl.BlockSpec((_CONV_ROWS, _C), lambda i: (i, 0))],
        )(hup_hw, fr_hw, w9, bias8)

    hits_all = jnp.concatenate(hits_list, axis=0)
    fhat_cf, loss_t, ppl_t = _call(
        _ppl_body, [(_B * _C, _S), (8, 128), (8, 128)], hits_all, f_cf)

    f_hat = fhat_cf.reshape(_B, _C, _HW, _HW)
    return (f_hat, loss_t[0, 0], ppl_t[0, 0])


# speedup vs baseline: 3.7087x; 1.5091x over previous
import jax
import jax.numpy as jnp
import numpy as np
from jax.experimental import pallas as pl
from jax.experimental.pallas import tpu as pltpu

_VOCAB = 4096
_B = 64
_C = 32
_HW = 16
_S = _HW * _HW
_N_FULL = _B * _S
_PNS = (1, 2, 4, 8, 16)
_PI = (0, 1, 1, 2)
_F32 = jnp.float32
_BF16 = jnp.bfloat16
_HIGH = jax.lax.Precision.HIGHEST
_CHUNK = 512
_CONV_ROWS = 2048
_NT = (((1,), (1,)), ((), ()))


def _pool_mat_t(pn: int) -> np.ndarray:
    k = _HW // pn
    p1 = np.zeros((pn, _HW), np.float32)
    for p in range(pn):
        p1[p, p * k:(p + 1) * k] = 1.0 / k
    p2 = np.kron(p1, p1)
    out = np.zeros((_S, 64), np.float32)
    out[:, :pn * pn] = p2.T
    return out


def _upsample_mat_t(pn: int):
    a = jax.image.resize(jnp.eye(pn, dtype=_F32), (_HW, pn), method="bicubic")
    u = jnp.kron(a, a)
    return jnp.zeros((64, _S), _F32).at[:pn * pn, :].set(u.T)


def _hw_of_cf(x_cf):
    return jnp.swapaxes(x_cf.reshape(_B, _C, _S), 1, 2).reshape(_N_FULL, _C)


def _cf_of_hw(x_hw):
    return jnp.swapaxes(x_hw.reshape(_B, _S, _C), 1, 2).reshape(_B * _C, _S)


def _nc_of_cf(x_cf, sq):
    return jnp.swapaxes(x_cf.reshape(_B, _C, sq), 1, 2).reshape(_B * sq, _C)


def _cf_of_nc(x_nc, sq):
    return jnp.swapaxes(x_nc.reshape(_B, sq, _C), 1, 2).reshape(_B * _C, sq)


def _hilo(x):
    hi = x.astype(_BF16)
    lo = (x - hi.astype(_F32)).astype(_BF16)
    return hi, lo


def _quant_chunk(rest, b96v, hilo, want_h):
    chunk = rest.shape[0]
    rh, rl = _hilo(rest)
    a96 = jnp.concatenate([rh, rh, rl], axis=1)
    sims = jax.lax.dot_general(a96, b96v, _NT, preferred_element_type=_F32)
    m = jnp.max(sims, axis=1, keepdims=True)
    onehot = (sims == m).astype(_BF16)
    h = None
    if want_h:
        hl = jax.lax.dot(onehot, hilo, preferred_element_type=_F32)
        h = hl[:, :_C] + hl[:, _C:]
    rows = jax.lax.broadcasted_iota(jnp.int32, (8, chunk), 0)
    cols = jax.lax.broadcasted_iota(jnp.int32, (8, chunk), 1)
    w8 = (cols // (chunk // 8) == rows).astype(_BF16)
    part = jax.lax.dot(w8, onehot, preferred_element_type=_F32)
    return h, part


def _conv_block(x, fr, w9, bias):
    n = x.shape[0]
    zeros = jnp.zeros_like(x)
    ii = jax.lax.broadcasted_iota(jnp.int32, (n, 1), 0)
    w = ii & (_HW - 1)
    h = (ii >> 4) & (_HW - 1)
    y = jnp.broadcast_to(bias, (n, _C)).astype(_F32)
    t = 0
    for dh in (-1, 0, 1):
        for dw in (-1, 0, 1):
            s = dh * _HW + dw
            if s == 0:
                xs = x
            elif s > 0:
                xs = jnp.concatenate([x[s:, :], zeros[:s, :]], axis=0)
            else:
                xs = jnp.concatenate([zeros[:(-s), :], x[:s, :]], axis=0)
            okh = jnp.logical_and(h + dh >= 0, h + dh < _HW)
            okw = jnp.logical_and(w + dw >= 0, w + dw < _HW)
            xm = jnp.where(jnp.logical_and(okh, okw), xs, 0.0)
            y = y + jax.lax.dot(xm.astype(_BF16), w9[t * _C:(t + 1) * _C, :],
                                preferred_element_type=_F32)
            t += 1
    return fr - 0.5 * x - 0.5 * y


def _body(f_cf_ref, emb_ref, w9s_ref, bias_ref, pts_ref, uts_ref,
          fhat_ref, loss_ref, ppl_ref,
          fr_hw, sc_h, sc_hup, sc_hits):
    emb = emb_ref[...]
    norm = jnp.sqrt(jnp.sum(emb * emb, axis=1, keepdims=True))
    en = emb / jnp.maximum(norm, 1e-12)
    ehi, elo = _hilo(en)
    b96v = jnp.concatenate([ehi, elo, ehi], axis=1)
    ghi, glo = _hilo(emb)
    hilo = jnp.concatenate([ghi, glo], axis=1)

    fr_hw[...] = _hw_of_cf(f_cf_ref[...])
    sc_hits[...] = jnp.zeros_like(sc_hits)

    for si, pn in enumerate(_PNS):
        sq = pn * pn
        n = _B * sq
        last = si == len(_PNS) - 1

        if last:
            def _qloop4(i, _):
                rest = fr_hw[pl.ds(i * _CHUNK, _CHUNK), :]
                _, part = _quant_chunk(rest, b96v, hilo, want_h=False)
                sc_hits[...] += part
                return 0
            jax.lax.fori_loop(0, _N_FULL // _CHUNK, _qloop4, 0)
            break

        fr_cf = f_cf_ref[...] if si == 0 else _cf_of_hw(fr_hw[...])
        psq = 4 if si == 0 else sq
        pt = pts_ref[si * _S:(si + 1) * _S, 0:psq]
        pooled_nc = _nc_of_cf(jax.lax.dot(fr_cf, pt, precision=_HIGH), psq)
        if pn == 1:
            r64 = jax.lax.broadcasted_iota(jnp.int32, (_B, 4 * _B), 0)
            c64 = jax.lax.broadcasted_iota(jnp.int32, (_B, 4 * _B), 1)
            m64 = jnp.where(c64 // 4 == r64, 0.25, 0.0)
            rest_nc = jax.lax.dot(m64, pooled_nc, precision=_HIGH)
        else:
            rest_nc = pooled_nc

        if n <= _CHUNK:
            h_nc, part = _quant_chunk(rest_nc, b96v, hilo, want_h=True)
            sc_hits[...] += part
        else:
            sc_h[pl.ds(0, n), :] = rest_nc

            def _qloop3(i, _):
                rest = sc_h[pl.ds(i * _CHUNK, _CHUNK), :]
                h, part = _quant_chunk(rest, b96v, hilo, want_h=True)
                sc_h[pl.ds(i * _CHUNK, _CHUNK), :] = h
                sc_hits[...] += part
                return 0
            jax.lax.fori_loop(0, n // _CHUNK, _qloop3, 0)
            h_nc = sc_h[pl.ds(0, n), :]

        if pn == 1:
            hup_hw = jnp.broadcast_to(
                h_nc.reshape(_B, 1, _C), (_B, _S, _C)).reshape(_N_FULL, _C)
        else:
            ut = uts_ref[si * 64:si * 64 + sq, :]
            hup_cf = jax.lax.dot(_cf_of_nc(h_nc, sq), ut, precision=_HIGH)
            hup_hw = _hw_of_cf(hup_cf)
        sc_hup[...] = hup_hw

        w9 = w9s_ref[si * 9 * _C:(si + 1) * 9 * _C, :].astype(_BF16)
        bias = bias_ref[si * 8:si * 8 + 1, :]

        def _cloop(j, _):
            rows = pl.ds(j * _CONV_ROWS, _CONV_ROWS)
            fr_hw[rows, :] = _conv_block(sc_hup[rows, :], fr_hw[rows, :],
                                         w9, bias)
            return 0
        jax.lax.fori_loop(0, _N_FULL // _CONV_ROWS, _cloop, 0)

    hits = jnp.sum(sc_hits[...], axis=0, keepdims=True)
    total = jnp.sum(hits)
    avg = hits / jnp.maximum(total, 1.0)
    ent = jnp.sum(avg * jnp.log(avg + 1e-10))
    f_cf = f_cf_ref[...]
    loss = 6.25 * (jnp.sum(f_cf * f_cf) / (_B * _C * _S))
    fhat_ref[...] = jnp.zeros_like(fhat_ref)
    loss_ref[...] = jnp.full((8, 128), loss, _F32)
    ppl_ref[...] = jnp.full((8, 128), jnp.exp(-ent), _F32)


def kernel(f_BChw, emb_weight, phi_w, phi_b):
    f_cf = f_BChw.astype(_F32).reshape(_B * _C, _S)

    pts = jnp.asarray(np.concatenate(
        [_pool_mat_t(2), _pool_mat_t(2), _pool_mat_t(4), _pool_mat_t(8)],
        axis=0))
    uts = jnp.concatenate([_upsample_mat_t(pn) for pn in (1, 2, 4, 8)],
                          axis=0)
    w9s = jnp.concatenate(
        [jnp.concatenate([phi_w[_PI[si], :, :, a, b].T
                          for a in range(3) for b in range(3)], axis=0)
         for si in range(4)], axis=0)
    bias = jnp.concatenate(
        [jnp.broadcast_to(phi_b[_PI[si]][None, :], (8, _C))
         for si in range(4)], axis=0)

    fhat_cf, loss_t, ppl_t = pl.pallas_call(
        _body,
        out_shape=[jax.ShapeDtypeStruct((_B * _C, _S), _F32),
                   jax.ShapeDtypeStruct((8, 128), _F32),
                   jax.ShapeDtypeStruct((8, 128), _F32)],
        scratch_shapes=[pltpu.VMEM((_N_FULL, _C), _F32),
                        pltpu.VMEM((_N_FULL, _C), _F32),
                        pltpu.VMEM((_N_FULL, _C), _F32),
                        pltpu.VMEM((8, _VOCAB), _F32)],
    )(f_cf, emb_weight, w9s, bias, pts, uts)

    return (fhat_cf.reshape(_B, _C, _HW, _HW), loss_t[0, 0], ppl_t[0, 0])
